# baseline (speedup 1.0000x reference)
import functools

import jax
import jax.numpy as jnp
from jax import lax
from jax.experimental import pallas as pl
from jax.experimental.pallas import tpu as pltpu
from jax.experimental.pallas import tpu_sc as plsc

N_NODES = 10000
N_EDGES = 160000
D = 256
DQ = 64

NC = 2
NS = 16
CHUNK = 128
NCHUNKS = N_EDGES // CHUNK
CPT = 79
CPT_LAST = NCHUNKS - 15 * CPT
N_PAD = 10240
STRIPE = N_PAD // NS
NDEG = 640
DSTRIPE = NDEG // NS


def _sc_aggregate(xf, src_c, dst_c):
    mesh = plsc.VectorSubcoreMesh(core_axis_name="c", subcore_axis_name="s")

    @functools.partial(
        pl.kernel,
        out_type=(
            jax.ShapeDtypeStruct((N_PAD, DQ), jnp.float32),
            jax.ShapeDtypeStruct((N_PAD, DQ), jnp.float32),
            jax.ShapeDtypeStruct((N_PAD, DQ), jnp.float32),
            jax.ShapeDtypeStruct((N_PAD, DQ), jnp.float32),
            jax.ShapeDtypeStruct((NDEG, 16), jnp.float32),
            jax.ShapeDtypeStruct((NDEG, 16), jnp.float32),
        ),
        mesh=mesh,
        compiler_params=pltpu.CompilerParams(use_tc_tiling_on_sc=False),
        scratch_types=[
            pltpu.VMEM((CPT, CHUNK), jnp.int32),
            pltpu.VMEM((CPT, CHUNK), jnp.int32),
            pltpu.VMEM((CPT, CHUNK), jnp.int32),
            pltpu.VMEM((CHUNK, DQ), jnp.float32),
            pltpu.VMEM((CHUNK, DQ), jnp.float32),
            pltpu.VMEM((CHUNK, 16), jnp.float32),
            pltpu.VMEM((CHUNK,), jnp.int32),
            pltpu.VMEM((32, DQ), jnp.float32),
            pltpu.VMEM((DSTRIPE, 16), jnp.float32),
            pltpu.VMEM_SHARED((N_PAD, DQ), jnp.float32),
            pltpu.VMEM_SHARED((NDEG, 16), jnp.float32),
            pltpu.SemaphoreType.DMA,
            pltpu.SemaphoreType.DMA,
            pltpu.SemaphoreType.DMA,
            pltpu.SemaphoreType.DMA,
            pltpu.SemaphoreType.DMA,
        ],
    )
    def k(xf_hbm, src_hbm, dst_hbm,
          a0_hbm, a1_hbm, a2_hbm, a3_hbm, deg0_hbm, deg1_hbm,
          src_v, dst_v, idx_v, rows_a, rows_b, onehot_v, rowidx_v,
          zb_v, zd_v, agg_sh, deg_sh, gsem0, gsem1, ssem0, ssem1, dsem):
        c = lax.axis_index("c")
        s = lax.axis_index("s")
        cnt = jnp.where(s == NS - 1, CPT_LAST, CPT)
        npairs = (cnt - 1) // 2

        zeros16 = jnp.zeros((16,), jnp.float32)
        ones16 = jnp.full((16,), 1.0, jnp.float32)
        iota16 = lax.iota(jnp.int32, 16)

        def init_zb(i, _):
            for kk in range(DQ // 16):
                zb_v[i, pl.ds(kk * 16, 16)] = zeros16
            return 0
        lax.fori_loop(0, 32, init_zb, 0)

        def init_zd(i, _):
            zd_v[i, :] = zeros16
            return 0
        lax.fori_loop(0, DSTRIPE, init_zd, 0)

        def zero_agg_stripe():
            def zero_one(r, _):
                pltpu.sync_copy(zb_v,
                                agg_sh.at[pl.ds(s * STRIPE + r * 32, 32)])
                return 0
            lax.fori_loop(0, STRIPE // 32, zero_one, 0)

        zero_agg_stripe()
        pltpu.sync_copy(zd_v, deg_sh.at[pl.ds(s * DSTRIPE, DSTRIPE)])

        def stage_full():
            pltpu.sync_copy(src_hbm.at[pl.ds(s * CPT, CPT)], src_v)
            pltpu.sync_copy(dst_hbm.at[pl.ds(s * CPT, CPT)], dst_v)

        def stage_last():
            pltpu.sync_copy(src_hbm.at[pl.ds(15 * CPT, CPT_LAST)],
                            src_v.at[pl.ds(0, CPT_LAST)])
            pltpu.sync_copy(dst_hbm.at[pl.ds(15 * CPT, CPT_LAST)],
                            dst_v.at[pl.ds(0, CPT_LAST)])

        pl.when(s < NS - 1)(stage_full)
        pl.when(s == NS - 1)(stage_last)

        def start_gather(j, buf, sem):
            pltpu.async_copy(xf_hbm.at[idx_v.at[j]], buf, sem)

        def wait_gather(j, buf, sem):
            pltpu.make_async_copy(xf_hbm.at[idx_v.at[j]], buf, sem).wait()

        def start_scat(j, buf, sem):
            pltpu.async_copy(buf, agg_sh.at[dst_v.at[j]], sem, add=True)

        def wait_scat(j, buf, sem):
            pltpu.make_async_copy(buf, agg_sh.at[dst_v.at[j]], sem).wait()

        def build_onehot(j):
            for g in range(CHUNK // 16):
                d16 = dst_v[j, pl.ds(g * 16, 16)]
                col = jnp.bitwise_and(d16, 15)
                rowidx_v[pl.ds(g * 16, 16)] = jnp.right_shift(d16, 4)
                for r in range(16):
                    onehot_v[g * 16 + r, :] = jnp.where(
                        iota16 == col[r], ones16, zeros16)

        def wait_deg():
            pltpu.make_async_copy(onehot_v, deg_sh.at[rowidx_v], dsem).wait()

        def run_pass(q, with_deg):
            def bld(r, _):
                for kk in range(CHUNK // 16):
                    sl = pl.ds(kk * 16, 16)
                    idx_v[r, sl] = jnp.left_shift(src_v[r, sl], 2) + q
                return 0
            lax.fori_loop(0, cnt, bld, 0)

            start_gather(0, rows_a, gsem0)
            start_gather(1, rows_b, gsem1)
            plsc.subcore_barrier()

            def pair(p, _):
                j0 = 2 * p
                j1 = j0 + 1
                wait_gather(j0, rows_a, gsem0)
                start_scat(j0, rows_a, ssem0)
                if with_deg:
                    jd = j0 + c
                    pl.when(p > 0)(wait_deg)
                    build_onehot(jd)
                    pltpu.async_copy(onehot_v, deg_sh.at[rowidx_v], dsem,
                                     add=True)
                wait_gather(j1, rows_b, gsem1)
                start_scat(j1, rows_b, ssem1)
                wait_scat(j0, rows_a, ssem0)
                pl.when(j0 + 2 < cnt)(
                    lambda: start_gather(j0 + 2, rows_a, gsem0))
                wait_scat(j1, rows_b, ssem1)
                pl.when(j1 + 2 < cnt)(
                    lambda: start_gather(j1 + 2, rows_b, gsem1))
                return 0
            lax.fori_loop(0, npairs, pair, 0)

            if with_deg:
                wait_deg()

            jl = cnt - 1
            wait_gather(jl, rows_a, gsem0)
            pltpu.sync_copy(rows_a, agg_sh.at[dst_v.at[jl]], add=True)
            if with_deg:
                def tail_deg():
                    build_onehot(jl)
                    pltpu.sync_copy(onehot_v, deg_sh.at[rowidx_v], add=True)
                pl.when(c == 0)(tail_deg)

            plsc.subcore_barrier()

        def copy_agg_out(aq_hbm):
            pltpu.sync_copy(agg_sh.at[pl.ds(s * STRIPE, STRIPE)],
                            aq_hbm.at[pl.ds(s * STRIPE, STRIPE)])

        def copy_deg_out(deg_hbm):
            pltpu.sync_copy(deg_sh.at[pl.ds(s * DSTRIPE, DSTRIPE)],
                            deg_hbm.at[pl.ds(s * DSTRIPE, DSTRIPE)])

        run_pass(2 * c, True)

        def out_c0():
            copy_agg_out(a0_hbm)
            copy_deg_out(deg0_hbm)

        def out_c1():
            copy_agg_out(a2_hbm)
            copy_deg_out(deg1_hbm)

        pl.when(c == 0)(out_c0)
        pl.when(c == 1)(out_c1)
        zero_agg_stripe()
        plsc.subcore_barrier()

        run_pass(2 * c + 1, False)
        pl.when(c == 0)(lambda: copy_agg_out(a1_hbm))
        pl.when(c == 1)(lambda: copy_agg_out(a3_hbm))

    return k(xf, src_c, dst_c)


def _tc_self_body(x_ref, ws_ref, o_ref):
    o_ref[:] = jnp.dot(x_ref[:], ws_ref[:], preferred_element_type=jnp.float32)


def _tc_self(x, w2_t):
    blk = 1000
    return pl.pallas_call(
        _tc_self_body,
        grid=(N_NODES // blk,),
        in_specs=[
            pl.BlockSpec((blk, D), lambda i: (i, 0)),
            pl.BlockSpec((D, D), lambda i: (0, 0)),
        ],
        out_specs=pl.BlockSpec((blk, D), lambda i: (i, 0)),
        out_shape=jax.ShapeDtypeStruct((N_NODES, D), jnp.float32),
    )(x, w2_t)


def _tc_body(a0_ref, a1_ref, a2_ref, a3_ref, d0_ref, d1_ref, out4_ref,
             w0_ref, w1_ref, w2_ref, w3_ref, o_ref):
    deg = jnp.maximum(d0_ref[:] + d1_ref[:], 1.0)
    acc = out4_ref[:]
    acc += jnp.dot(a0_ref[:] / deg, w0_ref[:],
                   preferred_element_type=jnp.float32)
    acc += jnp.dot(a1_ref[:] / deg, w1_ref[:],
                   preferred_element_type=jnp.float32)
    acc += jnp.dot(a2_ref[:] / deg, w2_ref[:],
                   preferred_element_type=jnp.float32)
    acc += jnp.dot(a3_ref[:] / deg, w3_ref[:],
                   preferred_element_type=jnp.float32)
    o_ref[:] = acc


def _tc_epilogue(aggs, deg0_col, deg1_col, out4, w1q_t):
    blk = 1000
    grid = (N_NODES // blk,)
    return pl.pallas_call(
        _tc_body,
        grid=grid,
        in_specs=[
            pl.BlockSpec((blk, DQ), lambda i: (i, 0)),
            pl.BlockSpec((blk, DQ), lambda i: (i, 0)),
            pl.BlockSpec((blk, DQ), lambda i: (i, 0)),
            pl.BlockSpec((blk, DQ), lambda i: (i, 0)),
            pl.BlockSpec((blk, 1), lambda i: (i, 0)),
            pl.BlockSpec((blk, 1), lambda i: (i, 0)),
            pl.BlockSpec((blk, D), lambda i: (i, 0)),
            pl.BlockSpec((DQ, D), lambda i: (0, 0)),
            pl.BlockSpec((DQ, D), lambda i: (0, 0)),
            pl.BlockSpec((DQ, D), lambda i: (0, 0)),
            pl.BlockSpec((DQ, D), lambda i: (0, 0)),
        ],
        out_specs=pl.BlockSpec((blk, D), lambda i: (i, 0)),
        out_shape=jax.ShapeDtypeStruct((N_NODES, D), jnp.float32),
    )(*aggs, deg0_col, deg1_col, out4, *w1q_t)


@jax.jit
def kernel(x, edge_index, W1, W2):
    src_c = edge_index[0].astype(jnp.int32).reshape(NCHUNKS, CHUNK)
    dst_c = edge_index[1].astype(jnp.int32).reshape(NCHUNKS, CHUNK)
    xf = x.reshape(N_NODES * 4, DQ)
    a0, a1, a2, a3, deg0, deg1 = _sc_aggregate(xf, src_c, dst_c)
    out4 = _tc_self(x, W2.T)
    deg0_col = deg0.reshape(-1)[:N_NODES].reshape(N_NODES, 1)
    deg1_col = deg1.reshape(-1)[:N_NODES].reshape(N_NODES, 1)
    w1q_t = [W1[:, i * DQ:(i + 1) * DQ].T for i in range(4)]
    return _tc_epilogue([a0, a1, a2, a3], deg0_col, deg1_col, out4, w1q_t)


# speedup vs baseline: 1.1446x; 1.1446x over previous
import functools

import jax
import jax.numpy as jnp
from jax import lax
from jax.experimental import pallas as pl
from jax.experimental.pallas import tpu as pltpu
from jax.experimental.pallas import tpu_sc as plsc

N_NODES = 10000
N_EDGES = 160000
D = 256
DQ = 64

NC = 2
NS = 16
CHUNK = 128
NCHUNKS = N_EDGES // CHUNK
CPT = 79
CPT_LAST = NCHUNKS - 15 * CPT
N_PAD = 10240
STRIPE = N_PAD // NS
NDEG = 640
DSTRIPE = NDEG // NS


def _sc_aggregate(xf, src_c, dst_c):
    mesh = plsc.VectorSubcoreMesh(core_axis_name="c", subcore_axis_name="s")

    @functools.partial(
        pl.kernel,
        out_type=(
            jax.ShapeDtypeStruct((N_PAD, DQ), jnp.float32),
            jax.ShapeDtypeStruct((N_PAD, DQ), jnp.float32),
            jax.ShapeDtypeStruct((N_PAD, DQ), jnp.float32),
            jax.ShapeDtypeStruct((N_PAD, DQ), jnp.float32),
            jax.ShapeDtypeStruct((NDEG, 16), jnp.float32),
            jax.ShapeDtypeStruct((NDEG, 16), jnp.float32),
        ),
        mesh=mesh,
        compiler_params=pltpu.CompilerParams(use_tc_tiling_on_sc=False),
        scratch_types=[
            pltpu.VMEM((CPT, CHUNK), jnp.int32),
            pltpu.VMEM((CPT, CHUNK), jnp.int32),
            pltpu.VMEM((CPT, CHUNK), jnp.int32),
            pltpu.VMEM((CHUNK, DQ), jnp.float32),
            pltpu.VMEM((CHUNK, DQ), jnp.float32),
            pltpu.VMEM((CHUNK, DQ), jnp.float32),
            pltpu.VMEM((CHUNK, DQ), jnp.float32),
            pltpu.VMEM((CHUNK, 16), jnp.float32),
            pltpu.VMEM((CHUNK,), jnp.int32),
            pltpu.VMEM((32, DQ), jnp.float32),
            pltpu.VMEM((DSTRIPE, 16), jnp.float32),
            pltpu.VMEM_SHARED((N_PAD, DQ), jnp.float32),
            pltpu.VMEM_SHARED((NDEG, 16), jnp.float32),
        ] + [pltpu.SemaphoreType.DMA] * 9,
    )
    def k(xf_hbm, src_hbm, dst_hbm,
          a0_hbm, a1_hbm, a2_hbm, a3_hbm, deg0_hbm, deg1_hbm,
          src_v, dst_v, idx_v, rows0, rows1, rows2, rows3,
          onehot_v, rowidx_v, zb_v, zd_v, agg_sh, deg_sh,
          g0, g1, g2, g3, s0, s1, s2, s3, dsem):
        R = [rows0, rows1, rows2, rows3]
        G = [g0, g1, g2, g3]
        S = [s0, s1, s2, s3]
        c = lax.axis_index("c")
        s = lax.axis_index("s")
        cnt = jnp.where(s == NS - 1, CPT_LAST, CPT)
        nquads = cnt // 4

        zeros16 = jnp.zeros((16,), jnp.float32)
        ones16 = jnp.full((16,), 1.0, jnp.float32)
        iota16 = lax.iota(jnp.int32, 16)

        def init_zb(i, _):
            for kk in range(DQ // 16):
                zb_v[i, pl.ds(kk * 16, 16)] = zeros16
            return 0
        lax.fori_loop(0, 32, init_zb, 0)

        def init_zd(i, _):
            zd_v[i, :] = zeros16
            return 0
        lax.fori_loop(0, DSTRIPE, init_zd, 0)

        def zero_agg_stripe():
            def zero_one(r, _):
                pltpu.sync_copy(zb_v,
                                agg_sh.at[pl.ds(s * STRIPE + r * 32, 32)])
                return 0
            lax.fori_loop(0, STRIPE // 32, zero_one, 0)

        zero_agg_stripe()
        pltpu.sync_copy(zd_v, deg_sh.at[pl.ds(s * DSTRIPE, DSTRIPE)])

        def stage_full():
            pltpu.sync_copy(src_hbm.at[pl.ds(s * CPT, CPT)], src_v)
            pltpu.sync_copy(dst_hbm.at[pl.ds(s * CPT, CPT)], dst_v)

        def stage_last():
            pltpu.sync_copy(src_hbm.at[pl.ds(15 * CPT, CPT_LAST)],
                            src_v.at[pl.ds(0, CPT_LAST)])
            pltpu.sync_copy(dst_hbm.at[pl.ds(15 * CPT, CPT_LAST)],
                            dst_v.at[pl.ds(0, CPT_LAST)])

        pl.when(s < NS - 1)(stage_full)
        pl.when(s == NS - 1)(stage_last)

        def start_gather(j, buf, sem):
            pltpu.async_copy(xf_hbm.at[idx_v.at[j]], buf, sem)

        def wait_gather(j, buf, sem):
            pltpu.make_async_copy(xf_hbm.at[idx_v.at[j]], buf, sem).wait()

        def start_scat(j, buf, sem):
            pltpu.async_copy(buf, agg_sh.at[dst_v.at[j]], sem, add=True)

        def wait_scat(j, buf, sem):
            pltpu.make_async_copy(buf, agg_sh.at[dst_v.at[j]], sem).wait()

        def build_onehot(j):
            for g in range(CHUNK // 16):
                d16 = dst_v[j, pl.ds(g * 16, 16)]
                col = jnp.bitwise_and(d16, 15)
                rowidx_v[pl.ds(g * 16, 16)] = jnp.right_shift(d16, 4)
                for r in range(16):
                    onehot_v[g * 16 + r, :] = jnp.where(
                        iota16 == col[r], ones16, zeros16)

        def wait_deg():
            pltpu.make_async_copy(onehot_v, deg_sh.at[rowidx_v], dsem).wait()

        def run_pass(q, with_deg):
            def bld(r, _):
                for kk in range(CHUNK // 16):
                    sl = pl.ds(kk * 16, 16)
                    idx_v[r, sl] = jnp.left_shift(src_v[r, sl], 2) + q
                return 0
            lax.fori_loop(0, cnt, bld, 0)

            for i in range(4):
                start_gather(i, R[i], G[i])
            plsc.subcore_barrier()

            def quad(p, _):
                base4 = 4 * p
                for i in range(4):
                    ji = base4 + i
                    wait_gather(ji, R[i], G[i])
                    start_scat(ji, R[i], S[i])
                    if with_deg:
                        def dg(ji=ji, first=(i < 2)):
                            if first:
                                pl.when(p > 0)(wait_deg)
                            else:
                                wait_deg()
                            build_onehot(ji)
                            pltpu.async_copy(onehot_v, deg_sh.at[rowidx_v],
                                             dsem, add=True)
                        pl.when((i & 1) == c)(dg)
                for i in range(4):
                    ji = base4 + i
                    wait_scat(ji, R[i], S[i])
                    pl.when(ji + 4 < cnt)(
                        lambda ji=ji, i=i: start_gather(ji + 4, R[i], G[i]))
                return 0
            lax.fori_loop(0, nquads, quad, 0)

            if with_deg:
                wait_deg()

            for r in range(3):
                def tail(r=r):
                    jt = 4 * nquads + r
                    wait_gather(jt, R[r], G[r])
                    pltpu.sync_copy(R[r], agg_sh.at[dst_v.at[jt]], add=True)
                    if with_deg:
                        def td():
                            build_onehot(jt)
                            pltpu.sync_copy(onehot_v, deg_sh.at[rowidx_v],
                                            add=True)
                        pl.when((r & 1) == c)(td)
                pl.when(4 * nquads + r < cnt)(tail)

            plsc.subcore_barrier()

        def copy_agg_out(aq_hbm):
            pltpu.sync_copy(agg_sh.at[pl.ds(s * STRIPE, STRIPE)],
                            aq_hbm.at[pl.ds(s * STRIPE, STRIPE)])

        def copy_deg_out(deg_hbm):
            pltpu.sync_copy(deg_sh.at[pl.ds(s * DSTRIPE, DSTRIPE)],
                            deg_hbm.at[pl.ds(s * DSTRIPE, DSTRIPE)])

        run_pass(2 * c, True)

        def out_c0():
            copy_agg_out(a0_hbm)
            copy_deg_out(deg0_hbm)

        def out_c1():
            copy_agg_out(a2_hbm)
            copy_deg_out(deg1_hbm)

        pl.when(c == 0)(out_c0)
        pl.when(c == 1)(out_c1)
        zero_agg_stripe()
        plsc.subcore_barrier()

        run_pass(2 * c + 1, False)
        pl.when(c == 0)(lambda: copy_agg_out(a1_hbm))
        pl.when(c == 1)(lambda: copy_agg_out(a3_hbm))

    return k(xf, src_c, dst_c)


def _tc_body(a0_ref, a1_ref, a2_ref, a3_ref, d0_ref, d1_ref, x_ref,
             w0_ref, w1_ref, w2_ref, w3_ref, ws_ref, o_ref):
    deg = jnp.maximum(d0_ref[:] + d1_ref[:], 1.0)
    acc = jnp.dot(x_ref[:], ws_ref[:], preferred_element_type=jnp.float32)
    acc += jnp.dot(a0_ref[:] / deg, w0_ref[:],
                   preferred_element_type=jnp.float32)
    acc += jnp.dot(a1_ref[:] / deg, w1_ref[:],
                   preferred_element_type=jnp.float32)
    acc += jnp.dot(a2_ref[:] / deg, w2_ref[:],
                   preferred_element_type=jnp.float32)
    acc += jnp.dot(a3_ref[:] / deg, w3_ref[:],
                   preferred_element_type=jnp.float32)
    o_ref[:] = acc


def _tc_epilogue(aggs, deg0_col, deg1_col, x, w1q_t, w2_t):
    blk = 1000
    grid = (N_NODES // blk,)
    return pl.pallas_call(
        _tc_body,
        grid=grid,
        in_specs=[
            pl.BlockSpec((blk, DQ), lambda i: (i, 0)),
            pl.BlockSpec((blk, DQ), lambda i: (i, 0)),
            pl.BlockSpec((blk, DQ), lambda i: (i, 0)),
            pl.BlockSpec((blk, DQ), lambda i: (i, 0)),
            pl.BlockSpec((blk, 1), lambda i: (i, 0)),
            pl.BlockSpec((blk, 1), lambda i: (i, 0)),
            pl.BlockSpec((blk, D), lambda i: (i, 0)),
            pl.BlockSpec((DQ, D), lambda i: (0, 0)),
            pl.BlockSpec((DQ, D), lambda i: (0, 0)),
            pl.BlockSpec((DQ, D), lambda i: (0, 0)),
            pl.BlockSpec((DQ, D), lambda i: (0, 0)),
            pl.BlockSpec((D, D), lambda i: (0, 0)),
        ],
        out_specs=pl.BlockSpec((blk, D), lambda i: (i, 0)),
        out_shape=jax.ShapeDtypeStruct((N_NODES, D), jnp.float32),
    )(*aggs, deg0_col, deg1_col, x, *w1q_t, w2_t)


@jax.jit
def kernel(x, edge_index, W1, W2):
    src_c = edge_index[0].astype(jnp.int32).reshape(NCHUNKS, CHUNK)
    dst_c = edge_index[1].astype(jnp.int32).reshape(NCHUNKS, CHUNK)
    xf = x.reshape(N_NODES * 4, DQ)
    a0, a1, a2, a3, deg0, deg1 = _sc_aggregate(xf, src_c, dst_c)
    deg0_col = deg0.reshape(-1)[:N_NODES].reshape(N_NODES, 1)
    deg1_col = deg1.reshape(-1)[:N_NODES].reshape(N_NODES, 1)
    w1q_t = [W1[:, i * DQ:(i + 1) * DQ].T for i in range(4)]
    return _tc_epilogue([a0, a1, a2, a3], deg0_col, deg1_col, x,
                        w1q_t, W2.T)


# speedup vs baseline: 1.1470x; 1.0021x over previous
import functools

import jax
import jax.numpy as jnp
from jax import lax
from jax.experimental import pallas as pl
from jax.experimental.pallas import tpu as pltpu
from jax.experimental.pallas import tpu_sc as plsc

N_NODES = 10000
N_EDGES = 160000
D = 256
DQ = 64

NC = 2
NS = 16
CHUNK = 128
NCHUNKS = N_EDGES // CHUNK
CPT = 79
CPT_LAST = NCHUNKS - 15 * CPT
N_PAD = 10240
STRIPE = N_PAD // NS
NDEG = 640
DSTRIPE = NDEG // NS


def _sc_aggregate(xf, edges_c):
    mesh = plsc.VectorSubcoreMesh(core_axis_name="c", subcore_axis_name="s")

    @functools.partial(
        pl.kernel,
        out_type=(
            jax.ShapeDtypeStruct((N_PAD, DQ), jnp.float32),
            jax.ShapeDtypeStruct((N_PAD, DQ), jnp.float32),
            jax.ShapeDtypeStruct((N_PAD, DQ), jnp.float32),
            jax.ShapeDtypeStruct((N_PAD, DQ), jnp.float32),
            jax.ShapeDtypeStruct((NDEG, 16), jnp.float32),
            jax.ShapeDtypeStruct((NDEG, 16), jnp.float32),
        ),
        mesh=mesh,
        compiler_params=pltpu.CompilerParams(use_tc_tiling_on_sc=False),
        scratch_types=[
            pltpu.VMEM((CPT, CHUNK), jnp.int32),
            pltpu.VMEM((CPT, CHUNK), jnp.int32),
            pltpu.VMEM((CPT, CHUNK), jnp.int32),
            pltpu.VMEM((CHUNK, DQ), jnp.float32),
            pltpu.VMEM((CHUNK, DQ), jnp.float32),
            pltpu.VMEM((CHUNK, DQ), jnp.float32),
            pltpu.VMEM((CHUNK, DQ), jnp.float32),
            pltpu.VMEM((CHUNK, 16), jnp.float32),
            pltpu.VMEM((CHUNK,), jnp.int32),
            pltpu.VMEM((32, DQ), jnp.float32),
            pltpu.VMEM((DSTRIPE, 16), jnp.float32),
            pltpu.VMEM_SHARED((N_PAD, DQ), jnp.float32),
            pltpu.VMEM_SHARED((NDEG, 16), jnp.float32),
        ] + [pltpu.SemaphoreType.DMA] * 9,
    )
    def k(xf_hbm, edges_hbm,
          a0_hbm, a1_hbm, a2_hbm, a3_hbm, deg0_hbm, deg1_hbm,
          src_v, dst_v, idx_v, rows0, rows1, rows2, rows3,
          onehot_v, rowidx_v, zb_v, zd_v, agg_sh, deg_sh,
          g0, g1, g2, g3, s0, s1, s2, s3, dsem):
        R = [rows0, rows1, rows2, rows3]
        G = [g0, g1, g2, g3]
        S = [s0, s1, s2, s3]
        c = lax.axis_index("c")
        s = lax.axis_index("s")
        cnt = jnp.where(s == NS - 1, CPT_LAST, CPT)
        nquads = cnt // 4

        zeros16 = jnp.zeros((16,), jnp.float32)
        ones16 = jnp.full((16,), 1.0, jnp.float32)
        iota16 = lax.iota(jnp.int32, 16)

        def init_zb(i, _):
            for kk in range(DQ // 16):
                zb_v[i, pl.ds(kk * 16, 16)] = zeros16
            return 0
        lax.fori_loop(0, 32, init_zb, 0)

        def init_zd(i, _):
            zd_v[i, :] = zeros16
            return 0
        lax.fori_loop(0, DSTRIPE, init_zd, 0)

        def zero_agg_stripe():
            def zero_one(r, _):
                pltpu.sync_copy(zb_v,
                                agg_sh.at[pl.ds(s * STRIPE + r * 32, 32)])
                return 0
            lax.fori_loop(0, STRIPE // 32, zero_one, 0)

        zero_agg_stripe()
        pltpu.sync_copy(zd_v, deg_sh.at[pl.ds(s * DSTRIPE, DSTRIPE)])

        def stage_full():
            pltpu.sync_copy(edges_hbm.at[0, pl.ds(s * CPT, CPT)], src_v)
            pltpu.sync_copy(edges_hbm.at[1, pl.ds(s * CPT, CPT)], dst_v)

        def stage_last():
            pltpu.sync_copy(edges_hbm.at[0, pl.ds(15 * CPT, CPT_LAST)],
                            src_v.at[pl.ds(0, CPT_LAST)])
            pltpu.sync_copy(edges_hbm.at[1, pl.ds(15 * CPT, CPT_LAST)],
                            dst_v.at[pl.ds(0, CPT_LAST)])

        pl.when(s < NS - 1)(stage_full)
        pl.when(s == NS - 1)(stage_last)

        def start_gather(j, buf, sem):
            pltpu.async_copy(xf_hbm.at[idx_v.at[j]], buf, sem)

        def wait_gather(j, buf, sem):
            pltpu.make_async_copy(xf_hbm.at[idx_v.at[j]], buf, sem).wait()

        def start_scat(j, buf, sem):
            pltpu.async_copy(buf, agg_sh.at[dst_v.at[j]], sem, add=True)

        def wait_scat(j, buf, sem):
            pltpu.make_async_copy(buf, agg_sh.at[dst_v.at[j]], sem).wait()

        def build_onehot(j):
            for g in range(CHUNK // 16):
                d16 = dst_v[j, pl.ds(g * 16, 16)]
                col = jnp.bitwise_and(d16, 15)
                rowidx_v[pl.ds(g * 16, 16)] = jnp.right_shift(d16, 4)
                for r in range(16):
                    onehot_v[g * 16 + r, :] = jnp.where(
                        iota16 == col[r], ones16, zeros16)

        def wait_deg():
            pltpu.make_async_copy(onehot_v, deg_sh.at[rowidx_v], dsem).wait()

        def run_pass(q, with_deg):
            def bld(r, _):
                for kk in range(CHUNK // 16):
                    sl = pl.ds(kk * 16, 16)
                    idx_v[r, sl] = jnp.left_shift(src_v[r, sl], 2) + q
                return 0
            lax.fori_loop(0, cnt, bld, 0)

            for i in range(4):
                start_gather(i, R[i], G[i])
            plsc.subcore_barrier()

            def quad(p, _):
                base4 = 4 * p
                for i in range(4):
                    ji = base4 + i
                    wait_gather(ji, R[i], G[i])
                    start_scat(ji, R[i], S[i])
                    if with_deg:
                        def dg(ji=ji, first=(i < 2)):
                            if first:
                                pl.when(p > 0)(wait_deg)
                            else:
                                wait_deg()
                            build_onehot(ji)
                            pltpu.async_copy(onehot_v, deg_sh.at[rowidx_v],
                                             dsem, add=True)
                        pl.when((i & 1) == c)(dg)
                for i in range(4):
                    ji = base4 + i
                    wait_scat(ji, R[i], S[i])
                    pl.when(ji + 4 < cnt)(
                        lambda ji=ji, i=i: start_gather(ji + 4, R[i], G[i]))
                return 0
            lax.fori_loop(0, nquads, quad, 0)

            if with_deg:
                wait_deg()

            for r in range(3):
                def tail(r=r):
                    jt = 4 * nquads + r
                    wait_gather(jt, R[r], G[r])
                    pltpu.sync_copy(R[r], agg_sh.at[dst_v.at[jt]], add=True)
                    if with_deg:
                        def td():
                            build_onehot(jt)
                            pltpu.sync_copy(onehot_v, deg_sh.at[rowidx_v],
                                            add=True)
                        pl.when((r & 1) == c)(td)
                pl.when(4 * nquads + r < cnt)(tail)

            plsc.subcore_barrier()

        def copy_agg_out(aq_hbm):
            pltpu.sync_copy(agg_sh.at[pl.ds(s * STRIPE, STRIPE)],
                            aq_hbm.at[pl.ds(s * STRIPE, STRIPE)])

        def copy_deg_out(deg_hbm):
            pltpu.sync_copy(deg_sh.at[pl.ds(s * DSTRIPE, DSTRIPE)],
                            deg_hbm.at[pl.ds(s * DSTRIPE, DSTRIPE)])

        run_pass(2 * c, True)

        def out_c0():
            copy_agg_out(a0_hbm)
            copy_deg_out(deg0_hbm)

        def out_c1():
            copy_agg_out(a2_hbm)
            copy_deg_out(deg1_hbm)

        pl.when(c == 0)(out_c0)
        pl.when(c == 1)(out_c1)
        zero_agg_stripe()
        plsc.subcore_barrier()

        run_pass(2 * c + 1, False)
        pl.when(c == 0)(lambda: copy_agg_out(a1_hbm))
        pl.when(c == 1)(lambda: copy_agg_out(a3_hbm))

    return k(xf, edges_c)


def _tc_body(a0_ref, a1_ref, a2_ref, a3_ref, d0_ref, d1_ref, x_ref,
             w0_ref, w1_ref, w2_ref, w3_ref, ws_ref, o_ref):
    deg = jnp.maximum(d0_ref[:] + d1_ref[:], 1.0)
    acc = jnp.dot(x_ref[:], ws_ref[:], preferred_element_type=jnp.float32)
    acc += jnp.dot(a0_ref[:] / deg, w0_ref[:],
                   preferred_element_type=jnp.float32)
    acc += jnp.dot(a1_ref[:] / deg, w1_ref[:],
                   preferred_element_type=jnp.float32)
    acc += jnp.dot(a2_ref[:] / deg, w2_ref[:],
                   preferred_element_type=jnp.float32)
    acc += jnp.dot(a3_ref[:] / deg, w3_ref[:],
                   preferred_element_type=jnp.float32)
    o_ref[:] = acc


def _tc_epilogue(aggs, deg0_col, deg1_col, x, w1q_t, w2_t):
    blk = 1000
    grid = (N_NODES // blk,)
    return pl.pallas_call(
        _tc_body,
        grid=grid,
        in_specs=[
            pl.BlockSpec((blk, DQ), lambda i: (i, 0)),
            pl.BlockSpec((blk, DQ), lambda i: (i, 0)),
            pl.BlockSpec((blk, DQ), lambda i: (i, 0)),
            pl.BlockSpec((blk, DQ), lambda i: (i, 0)),
            pl.BlockSpec((blk, 1), lambda i: (i, 0)),
            pl.BlockSpec((blk, 1), lambda i: (i, 0)),
            pl.BlockSpec((blk, D), lambda i: (i, 0)),
            pl.BlockSpec((DQ, D), lambda i: (0, 0)),
            pl.BlockSpec((DQ, D), lambda i: (0, 0)),
            pl.BlockSpec((DQ, D), lambda i: (0, 0)),
            pl.BlockSpec((DQ, D), lambda i: (0, 0)),
            pl.BlockSpec((D, D), lambda i: (0, 0)),
        ],
        out_specs=pl.BlockSpec((blk, D), lambda i: (i, 0)),
        out_shape=jax.ShapeDtypeStruct((N_NODES, D), jnp.float32),
    )(*aggs, deg0_col, deg1_col, x, *w1q_t, w2_t)


@jax.jit
def kernel(x, edge_index, W1, W2):
    edges_c = edge_index.astype(jnp.int32).reshape(2, NCHUNKS, CHUNK)
    xf = x.reshape(N_NODES * 4, DQ)
    a0, a1, a2, a3, deg0, deg1 = _sc_aggregate(xf, edges_c)
    deg0_col = deg0.reshape(-1)[:N_NODES].reshape(N_NODES, 1)
    deg1_col = deg1.reshape(-1)[:N_NODES].reshape(N_NODES, 1)
    w1q_t = [W1[:, i * DQ:(i + 1) * DQ].T for i in range(4)]
    return _tc_epilogue([a0, a1, a2, a3], deg0_col, deg1_col, x,
                        w1q_t, W2.T)


# speedup vs baseline: 1.2460x; 1.0863x over previous
import functools

import jax
import jax.numpy as jnp
from jax import lax
from jax.experimental import pallas as pl
from jax.experimental.pallas import tpu as pltpu
from jax.experimental.pallas import tpu_sc as plsc

N_NODES = 10000
N_EDGES = 160000
D = 256
DQ = 64

NC = 2
NS = 16
CHUNK = 128
NCHUNKS = N_EDGES // CHUNK
CPT = 79
CPT_LAST = NCHUNKS - 15 * CPT
N_PAD = 10240
STRIPE = N_PAD // NS
DSTRIPE = 40


def _sc_aggregate(xf, src_c, dst_c):
    mesh = plsc.VectorSubcoreMesh(core_axis_name="c", subcore_axis_name="s")

    @functools.partial(
        pl.kernel,
        out_type=(
            jax.ShapeDtypeStruct((N_PAD, D), jnp.float32),
            jax.ShapeDtypeStruct((N_PAD, 16), jnp.float32),
            jax.ShapeDtypeStruct((N_PAD, 16), jnp.float32),
        ),
        mesh=mesh,
        compiler_params=pltpu.CompilerParams(use_tc_tiling_on_sc=False),
        scratch_types=[
            pltpu.VMEM((CPT, CHUNK), jnp.int32),
            pltpu.VMEM((CPT, CHUNK), jnp.int32),
            pltpu.VMEM((CPT, CHUNK), jnp.int32),
            pltpu.VMEM((CHUNK, DQ), jnp.float32),
            pltpu.VMEM((CHUNK, DQ), jnp.float32),
            pltpu.VMEM((CHUNK, DQ), jnp.float32),
            pltpu.VMEM((CHUNK, DQ), jnp.float32),
            pltpu.VMEM((CHUNK, 16), jnp.float32),
            pltpu.VMEM((32, DQ), jnp.float32),
            pltpu.VMEM((DSTRIPE, 16), jnp.float32),
            pltpu.VMEM_SHARED((N_PAD, DQ), jnp.float32),
            pltpu.VMEM_SHARED((N_PAD, 16), jnp.float32),
        ] + [pltpu.SemaphoreType.DMA] * 9,
    )
    def k(xf_hbm, src_hbm, dst_hbm,
          agg_hbm, deg0_hbm, deg1_hbm,
          src_v, dst_v, idx_v, rows0, rows1, rows2, rows3,
          ones_v, zb_v, zd_v, agg_sh, deg_sh,
          g0, g1, g2, g3, s0, s1, s2, s3, dsem):
        R = [rows0, rows1, rows2, rows3]
        G = [g0, g1, g2, g3]
        S = [s0, s1, s2, s3]
        c = lax.axis_index("c")
        s = lax.axis_index("s")
        cnt = jnp.where(s == NS - 1, CPT_LAST, CPT)
        nquads = cnt // 4

        zeros16 = jnp.zeros((16,), jnp.float32)
        ones16 = jnp.full((16,), 1.0, jnp.float32)

        def init_zb(i, _):
            for kk in range(DQ // 16):
                zb_v[i, pl.ds(kk * 16, 16)] = zeros16
            return 0
        lax.fori_loop(0, 32, init_zb, 0)

        def init_zd(i, _):
            zd_v[i, :] = zeros16
            return 0
        lax.fori_loop(0, DSTRIPE, init_zd, 0)

        def init_ones(i, _):
            ones_v[i, :] = ones16
            return 0
        lax.fori_loop(0, CHUNK, init_ones, 0)

        def zero_agg_stripe():
            def zero_one(r, _):
                pltpu.sync_copy(zb_v,
                                agg_sh.at[pl.ds(s * STRIPE + r * 32, 32)])
                return 0
            lax.fori_loop(0, STRIPE // 32, zero_one, 0)

        zero_agg_stripe()

        def zero_deg(r, _):
            pltpu.sync_copy(
                zd_v, deg_sh.at[pl.ds(s * STRIPE + r * DSTRIPE, DSTRIPE)])
            return 0
        lax.fori_loop(0, STRIPE // DSTRIPE, zero_deg, 0)

        def stage_full():
            pltpu.sync_copy(src_hbm.at[pl.ds(s * CPT, CPT)], src_v)
            pltpu.sync_copy(dst_hbm.at[pl.ds(s * CPT, CPT)], dst_v)

        def stage_last():
            pltpu.sync_copy(src_hbm.at[pl.ds(15 * CPT, CPT_LAST)],
                            src_v.at[pl.ds(0, CPT_LAST)])
            pltpu.sync_copy(dst_hbm.at[pl.ds(15 * CPT, CPT_LAST)],
                            dst_v.at[pl.ds(0, CPT_LAST)])

        pl.when(s < NS - 1)(stage_full)
        pl.when(s == NS - 1)(stage_last)

        def start_gather(j, buf, sem):
            pltpu.async_copy(xf_hbm.at[idx_v.at[j]], buf, sem)

        def wait_gather(j, buf, sem):
            pltpu.make_async_copy(xf_hbm.at[idx_v.at[j]], buf, sem).wait()

        def start_scat(j, buf, sem):
            pltpu.async_copy(buf, agg_sh.at[dst_v.at[j]], sem, add=True)

        def wait_scat(j, buf, sem):
            pltpu.make_async_copy(buf, agg_sh.at[dst_v.at[j]], sem).wait()

        def wait_deg(j):
            pltpu.make_async_copy(
                ones_v, deg_sh.at[dst_v.at[j]], dsem).wait()

        def run_pass(q, with_deg):
            def bld(r, _):
                for kk in range(CHUNK // 16):
                    sl = pl.ds(kk * 16, 16)
                    idx_v[r, sl] = jnp.left_shift(src_v[r, sl], 2) + q
                return 0
            lax.fori_loop(0, cnt, bld, 0)

            for i in range(4):
                start_gather(i, R[i], G[i])
            plsc.subcore_barrier()

            def quad(p, _):
                base4 = 4 * p
                for i in range(4):
                    ji = base4 + i
                    wait_gather(ji, R[i], G[i])
                    start_scat(ji, R[i], S[i])
                    if with_deg:
                        def dg(ji=ji, first=(i < 2)):
                            if first:
                                pl.when(p > 0)(lambda: wait_deg(ji))
                            else:
                                wait_deg(ji)
                            pltpu.async_copy(ones_v, deg_sh.at[dst_v.at[ji]],
                                             dsem, add=True)
                        pl.when((i & 1) == c)(dg)
                for i in range(4):
                    ji = base4 + i
                    wait_scat(ji, R[i], S[i])
                    pl.when(ji + 4 < cnt)(
                        lambda ji=ji, i=i: start_gather(ji + 4, R[i], G[i]))
                return 0
            lax.fori_loop(0, nquads, quad, 0)

            if with_deg:
                wait_deg(0)

            for r in range(3):
                def tail(r=r):
                    jt = 4 * nquads + r
                    wait_gather(jt, R[r], G[r])
                    pltpu.sync_copy(R[r], agg_sh.at[dst_v.at[jt]], add=True)
                    if with_deg:
                        def td():
                            pltpu.sync_copy(ones_v, deg_sh.at[dst_v.at[jt]],
                                            add=True)
                        pl.when((r & 1) == c)(td)
                pl.when(4 * nquads + r < cnt)(tail)

            plsc.subcore_barrier()

        def copy_agg_out(q):
            pltpu.sync_copy(agg_sh.at[pl.ds(s * STRIPE, STRIPE)],
                            agg_hbm.at[pl.ds(s * STRIPE, STRIPE),
                                       pl.ds(q * DQ, DQ)])

        def copy_deg_out(deg_hbm):
            pltpu.sync_copy(deg_sh.at[pl.ds(s * STRIPE, STRIPE)],
                            deg_hbm.at[pl.ds(s * STRIPE, STRIPE)])

        run_pass(2 * c, True)

        copy_agg_out(2 * c)
        pl.when(c == 0)(lambda: copy_deg_out(deg0_hbm))
        pl.when(c == 1)(lambda: copy_deg_out(deg1_hbm))
        zero_agg_stripe()
        plsc.subcore_barrier()

        run_pass(2 * c + 1, False)
        copy_agg_out(2 * c + 1)

    return k(xf, src_c, dst_c)


def _tc_body(a_ref, d0_ref, d1_ref, x_ref, w1_ref, ws_ref, o_ref):
    deg = jnp.maximum(d0_ref[:, :1] + d1_ref[:, :1], 1.0)
    acc = jnp.dot(x_ref[:], ws_ref[:], preferred_element_type=jnp.float32)
    acc += jnp.dot(a_ref[:], w1_ref[:],
                   preferred_element_type=jnp.float32) / deg
    o_ref[:] = acc


def _tc_epilogue(agg, deg0, deg1, x, w1_t, w2_t):
    blk = 2000
    grid = (N_NODES // blk,)
    return pl.pallas_call(
        _tc_body,
        grid=grid,
        in_specs=[
            pl.BlockSpec((blk, D), lambda i: (i, 0)),
            pl.BlockSpec((blk, 16), lambda i: (i, 0)),
            pl.BlockSpec((blk, 16), lambda i: (i, 0)),
            pl.BlockSpec((blk, D), lambda i: (i, 0)),
            pl.BlockSpec((D, D), lambda i: (0, 0)),
            pl.BlockSpec((D, D), lambda i: (0, 0)),
        ],
        out_specs=pl.BlockSpec((blk, D), lambda i: (i, 0)),
        out_shape=jax.ShapeDtypeStruct((N_NODES, D), jnp.float32),
    )(agg, deg0, deg1, x, w1_t, w2_t)


@jax.jit
def kernel(x, edge_index, W1, W2):
    src_c = edge_index[0].astype(jnp.int32).reshape(NCHUNKS, CHUNK)
    dst_c = edge_index[1].astype(jnp.int32).reshape(NCHUNKS, CHUNK)
    xf = x.reshape(N_NODES * 4, DQ)
    agg, deg0, deg1 = _sc_aggregate(xf, src_c, dst_c)
    return _tc_epilogue(agg, deg0, deg1, x, W1.T, W2.T)


# speedup vs baseline: 1.2471x; 1.0009x over previous
import functools

import jax
import jax.numpy as jnp
from jax import lax
from jax.experimental import pallas as pl
from jax.experimental.pallas import tpu as pltpu
from jax.experimental.pallas import tpu_sc as plsc

N_NODES = 10000
N_EDGES = 160000
D = 256
DQ = 64

NC = 2
NS = 16
CHUNK = 128
NCHUNKS = N_EDGES // CHUNK
CPT = 79
CPT_LAST = NCHUNKS - 15 * CPT
N_PAD = 10240
STRIPE = N_PAD // NS
DSTRIPE = 40


def _sc_aggregate(xf, edges_c):
    mesh = plsc.VectorSubcoreMesh(core_axis_name="c", subcore_axis_name="s")

    @functools.partial(
        pl.kernel,
        out_type=(
            jax.ShapeDtypeStruct((N_PAD, D), jnp.float32),
            jax.ShapeDtypeStruct((N_PAD, 16), jnp.float32),
            jax.ShapeDtypeStruct((N_PAD, 16), jnp.float32),
        ),
        mesh=mesh,
        compiler_params=pltpu.CompilerParams(use_tc_tiling_on_sc=False),
        scratch_types=[
            pltpu.VMEM((CPT, CHUNK), jnp.int32),
            pltpu.VMEM((CPT, CHUNK), jnp.int32),
            pltpu.VMEM((CPT, CHUNK), jnp.int32),
            pltpu.VMEM((CHUNK, DQ), jnp.float32),
            pltpu.VMEM((CHUNK, DQ), jnp.float32),
            pltpu.VMEM((CHUNK, DQ), jnp.float32),
            pltpu.VMEM((CHUNK, DQ), jnp.float32),
            pltpu.VMEM((CHUNK, 16), jnp.float32),
            pltpu.VMEM((32, DQ), jnp.float32),
            pltpu.VMEM((DSTRIPE, 16), jnp.float32),
            pltpu.VMEM_SHARED((N_PAD, DQ), jnp.float32),
            pltpu.VMEM_SHARED((N_PAD, 16), jnp.float32),
        ] + [pltpu.SemaphoreType.DMA] * 9,
    )
    def k(xf_hbm, edges_hbm,
          agg_hbm, deg0_hbm, deg1_hbm,
          src_v, dst_v, idx_v, rows0, rows1, rows2, rows3,
          ones_v, zb_v, zd_v, agg_sh, deg_sh,
          g0, g1, g2, g3, s0, s1, s2, s3, dsem):
        R = [rows0, rows1, rows2, rows3]
        G = [g0, g1, g2, g3]
        S = [s0, s1, s2, s3]
        c = lax.axis_index("c")
        s = lax.axis_index("s")
        cnt = jnp.where(s == NS - 1, CPT_LAST, CPT)
        nquads = cnt // 4

        zeros16 = jnp.zeros((16,), jnp.float32)
        ones16 = jnp.full((16,), 1.0, jnp.float32)

        def init_zb(i, _):
            for kk in range(DQ // 16):
                zb_v[i, pl.ds(kk * 16, 16)] = zeros16
            return 0
        lax.fori_loop(0, 32, init_zb, 0)

        def init_zd(i, _):
            zd_v[i, :] = zeros16
            return 0
        lax.fori_loop(0, DSTRIPE, init_zd, 0)

        def init_ones(i, _):
            ones_v[i, :] = ones16
            return 0
        lax.fori_loop(0, CHUNK, init_ones, 0)

        def zero_agg_stripe():
            def zero_one(r, _):
                pltpu.sync_copy(zb_v,
                                agg_sh.at[pl.ds(s * STRIPE + r * 32, 32)])
                return 0
            lax.fori_loop(0, STRIPE // 32, zero_one, 0)

        zero_agg_stripe()

        def zero_deg(r, _):
            pltpu.sync_copy(
                zd_v, deg_sh.at[pl.ds(s * STRIPE + r * DSTRIPE, DSTRIPE)])
            return 0
        lax.fori_loop(0, STRIPE // DSTRIPE, zero_deg, 0)

        def stage_full():
            pltpu.sync_copy(edges_hbm.at[pl.ds(s * CPT, CPT)], src_v)
            pltpu.sync_copy(edges_hbm.at[pl.ds(NCHUNKS + s * CPT, CPT)],
                            dst_v)

        def stage_last():
            pltpu.sync_copy(edges_hbm.at[pl.ds(15 * CPT, CPT_LAST)],
                            src_v.at[pl.ds(0, CPT_LAST)])
            pltpu.sync_copy(edges_hbm.at[pl.ds(NCHUNKS + 15 * CPT, CPT_LAST)],
                            dst_v.at[pl.ds(0, CPT_LAST)])

        pl.when(s < NS - 1)(stage_full)
        pl.when(s == NS - 1)(stage_last)

        def start_gather(j, buf, sem):
            pltpu.async_copy(xf_hbm.at[idx_v.at[j]], buf, sem)

        def wait_gather(j, buf, sem):
            pltpu.make_async_copy(xf_hbm.at[idx_v.at[j]], buf, sem).wait()

        def start_scat(j, buf, sem):
            pltpu.async_copy(buf, agg_sh.at[dst_v.at[j]], sem, add=True)

        def wait_scat(j, buf, sem):
            pltpu.make_async_copy(buf, agg_sh.at[dst_v.at[j]], sem).wait()

        def wait_deg(j):
            pltpu.make_async_copy(
                ones_v, deg_sh.at[dst_v.at[j]], dsem).wait()

        def run_pass(q, with_deg):
            def bld(r, _):
                for kk in range(CHUNK // 16):
                    sl = pl.ds(kk * 16, 16)
                    idx_v[r, sl] = jnp.left_shift(src_v[r, sl], 2) + q
                return 0
            lax.fori_loop(0, cnt, bld, 0)

            for i in range(4):
                start_gather(i, R[i], G[i])
            plsc.subcore_barrier()

            def quad(p, _):
                base4 = 4 * p
                for i in range(4):
                    ji = base4 + i
                    wait_gather(ji, R[i], G[i])
                    start_scat(ji, R[i], S[i])
                    if with_deg:
                        def dg(ji=ji, first=(i < 2)):
                            if first:
                                pl.when(p > 0)(lambda: wait_deg(ji))
                            else:
                                wait_deg(ji)
                            pltpu.async_copy(ones_v, deg_sh.at[dst_v.at[ji]],
                                             dsem, add=True)
                        pl.when((i & 1) == c)(dg)
                for i in range(4):
                    ji = base4 + i
                    wait_scat(ji, R[i], S[i])
                    pl.when(ji + 4 < cnt)(
                        lambda ji=ji, i=i: start_gather(ji + 4, R[i], G[i]))
                return 0
            lax.fori_loop(0, nquads, quad, 0)

            if with_deg:
                wait_deg(0)

            for r in range(3):
                def tail(r=r):
                    jt = 4 * nquads + r
                    wait_gather(jt, R[r], G[r])
                    pltpu.sync_copy(R[r], agg_sh.at[dst_v.at[jt]], add=True)
                    if with_deg:
                        def td():
                            pltpu.sync_copy(ones_v, deg_sh.at[dst_v.at[jt]],
                                            add=True)
                        pl.when((r & 1) == c)(td)
                pl.when(4 * nquads + r < cnt)(tail)

            plsc.subcore_barrier()

        def copy_agg_out(q):
            pltpu.sync_copy(agg_sh.at[pl.ds(s * STRIPE, STRIPE)],
                            agg_hbm.at[pl.ds(s * STRIPE, STRIPE),
                                       pl.ds(q * DQ, DQ)])

        def copy_deg_out(deg_hbm):
            pltpu.sync_copy(deg_sh.at[pl.ds(s * STRIPE, STRIPE)],
                            deg_hbm.at[pl.ds(s * STRIPE, STRIPE)])

        run_pass(2 * c, True)

        copy_agg_out(2 * c)
        pl.when(c == 0)(lambda: copy_deg_out(deg0_hbm))
        pl.when(c == 1)(lambda: copy_deg_out(deg1_hbm))
        zero_agg_stripe()
        plsc.subcore_barrier()

        run_pass(2 * c + 1, False)
        copy_agg_out(2 * c + 1)

    return k(xf, edges_c)


def _tc_body(a_ref, d0_ref, d1_ref, x_ref, w1_ref, ws_ref, o_ref):
    deg = jnp.maximum(d0_ref[:, :1] + d1_ref[:, :1], 1.0)
    acc = jnp.dot(x_ref[:], ws_ref[:], preferred_element_type=jnp.float32)
    acc += jnp.dot(a_ref[:], w1_ref[:],
                   preferred_element_type=jnp.float32) / deg
    o_ref[:] = acc


def _tc_epilogue(agg, deg0, deg1, x, w1_t, w2_t):
    blk = 2000
    grid = (N_NODES // blk,)
    return pl.pallas_call(
        _tc_body,
        grid=grid,
        in_specs=[
            pl.BlockSpec((blk, D), lambda i: (i, 0)),
            pl.BlockSpec((blk, 16), lambda i: (i, 0)),
            pl.BlockSpec((blk, 16), lambda i: (i, 0)),
            pl.BlockSpec((blk, D), lambda i: (i, 0)),
            pl.BlockSpec((D, D), lambda i: (0, 0)),
            pl.BlockSpec((D, D), lambda i: (0, 0)),
        ],
        out_specs=pl.BlockSpec((blk, D), lambda i: (i, 0)),
        out_shape=jax.ShapeDtypeStruct((N_NODES, D), jnp.float32),
    )(agg, deg0, deg1, x, w1_t, w2_t)


@jax.jit
def kernel(x, edge_index, W1, W2):
    edges_c = edge_index.astype(jnp.int32).reshape(2 * NCHUNKS, CHUNK)
    xf = x.reshape(N_NODES * 4, DQ)
    agg, deg0, deg1 = _sc_aggregate(xf, edges_c)
    return _tc_epilogue(agg, deg0, deg1, x, W1.T, W2.T)


# speedup vs baseline: 1.2652x; 1.0145x over previous
import functools

import jax
import jax.numpy as jnp
from jax import lax
from jax.experimental import pallas as pl
from jax.experimental.pallas import tpu as pltpu
from jax.experimental.pallas import tpu_sc as plsc

N_NODES = 10000
N_EDGES = 160000
D = 256
DQ = 64

NC = 2
NS = 16
CHUNK = 128
NCHUNKS = N_EDGES // CHUNK
CPT = 79
CPT_LAST = NCHUNKS - 15 * CPT
N_PAD = 10240
STRIPE = N_PAD // NS
DSTRIPE = 40


def _sc_aggregate(xf, edges_c):
    mesh = plsc.VectorSubcoreMesh(core_axis_name="c", subcore_axis_name="s")

    @functools.partial(
        pl.kernel,
        out_type=(
            jax.ShapeDtypeStruct((N_PAD, D), jnp.float32),
            jax.ShapeDtypeStruct((N_PAD, 16), jnp.float32),
            jax.ShapeDtypeStruct((N_PAD, 16), jnp.float32),
        ),
        mesh=mesh,
        compiler_params=pltpu.CompilerParams(use_tc_tiling_on_sc=False),
        scratch_types=[
            pltpu.VMEM((CPT, CHUNK), jnp.int32),
            pltpu.VMEM((CPT, CHUNK), jnp.int32),
            pltpu.VMEM((CHUNK, DQ), jnp.float32),
            pltpu.VMEM((CHUNK, DQ), jnp.float32),
            pltpu.VMEM((CHUNK, DQ), jnp.float32),
            pltpu.VMEM((CHUNK, DQ), jnp.float32),
            pltpu.VMEM((CHUNK, DQ), jnp.float32),
            pltpu.VMEM((CHUNK, DQ), jnp.float32),
            pltpu.VMEM((CHUNK, 16), jnp.float32),
            pltpu.VMEM((32, DQ), jnp.float32),
            pltpu.VMEM((DSTRIPE, 16), jnp.float32),
            pltpu.VMEM_SHARED((N_PAD, DQ), jnp.float32),
            pltpu.VMEM_SHARED((N_PAD, 16), jnp.float32),
        ] + [pltpu.SemaphoreType.DMA] * 13,
    )
    def k(xf_hbm, edges_hbm,
          agg_hbm, deg0_hbm, deg1_hbm,
          idx_v, dst_v, rows0, rows1, rows2, rows3, rows4, rows5,
          ones_v, zb_v, zd_v, agg_sh, deg_sh,
          g0, g1, g2, g3, g4, g5, s0, s1, s2, s3, s4, s5, dsem):
        R = [rows0, rows1, rows2, rows3, rows4, rows5]
        G = [g0, g1, g2, g3, g4, g5]
        S = [s0, s1, s2, s3, s4, s5]
        c = lax.axis_index("c")
        s = lax.axis_index("s")
        cnt = jnp.where(s == NS - 1, CPT_LAST, CPT)
        NB = 6
        nrounds = cnt // NB

        zeros16 = jnp.zeros((16,), jnp.float32)
        ones16 = jnp.full((16,), 1.0, jnp.float32)

        def init_zb(i, _):
            for kk in range(DQ // 16):
                zb_v[i, pl.ds(kk * 16, 16)] = zeros16
            return 0
        lax.fori_loop(0, 32, init_zb, 0)

        def init_zd(i, _):
            zd_v[i, :] = zeros16
            return 0
        lax.fori_loop(0, DSTRIPE, init_zd, 0)

        def init_ones(i, _):
            ones_v[i, :] = ones16
            return 0
        lax.fori_loop(0, CHUNK, init_ones, 0)

        def zero_agg_stripe():
            def zero_one(r, _):
                pltpu.sync_copy(zb_v,
                                agg_sh.at[pl.ds(s * STRIPE + r * 32, 32)])
                return 0
            lax.fori_loop(0, STRIPE // 32, zero_one, 0)

        zero_agg_stripe()

        def zero_deg(r, _):
            pltpu.sync_copy(
                zd_v, deg_sh.at[pl.ds(s * STRIPE + r * DSTRIPE, DSTRIPE)])
            return 0
        lax.fori_loop(0, STRIPE // DSTRIPE, zero_deg, 0)

        def stage_full():
            pltpu.sync_copy(edges_hbm.at[pl.ds(s * CPT, CPT)], idx_v)
            pltpu.sync_copy(edges_hbm.at[pl.ds(NCHUNKS + s * CPT, CPT)],
                            dst_v)

        def stage_last():
            pltpu.sync_copy(edges_hbm.at[pl.ds(15 * CPT, CPT_LAST)],
                            idx_v.at[pl.ds(0, CPT_LAST)])
            pltpu.sync_copy(edges_hbm.at[pl.ds(NCHUNKS + 15 * CPT, CPT_LAST)],
                            dst_v.at[pl.ds(0, CPT_LAST)])

        pl.when(s < NS - 1)(stage_full)
        pl.when(s == NS - 1)(stage_last)

        def start_gather(j, buf, sem):
            pltpu.async_copy(xf_hbm.at[idx_v.at[j]], buf, sem)

        def wait_gather(j, buf, sem):
            pltpu.make_async_copy(xf_hbm.at[idx_v.at[j]], buf, sem).wait()

        def start_scat(j, buf, sem):
            pltpu.async_copy(buf, agg_sh.at[dst_v.at[j]], sem, add=True)

        def wait_scat(j, buf, sem):
            pltpu.make_async_copy(buf, agg_sh.at[dst_v.at[j]], sem).wait()

        def wait_deg(j):
            pltpu.make_async_copy(
                ones_v, deg_sh.at[dst_v.at[j]], dsem).wait()

        def run_pass(q, first_pass, with_deg):
            def bld(r, _):
                for kk in range(CHUNK // 16):
                    sl = pl.ds(kk * 16, 16)
                    if first_pass:
                        idx_v[r, sl] = jnp.left_shift(idx_v[r, sl], 2) + q
                    else:
                        idx_v[r, sl] = idx_v[r, sl] + 1
                return 0
            lax.fori_loop(0, cnt, bld, 0)

            for i in range(NB):
                start_gather(i, R[i], G[i])
            plsc.subcore_barrier()

            def rnd(p, _):
                base = NB * p
                for i in range(NB):
                    ji = base + i
                    wait_gather(ji, R[i], G[i])
                    start_scat(ji, R[i], S[i])
                    if with_deg:
                        def dg(ji=ji, first=(i < 2)):
                            if first:
                                pl.when(p > 0)(lambda: wait_deg(ji))
                            else:
                                wait_deg(ji)
                            pltpu.async_copy(ones_v, deg_sh.at[dst_v.at[ji]],
                                             dsem, add=True)
                        pl.when((i & 1) == c)(dg)
                for i in range(NB):
                    ji = base + i
                    wait_scat(ji, R[i], S[i])
                    pl.when(ji + NB < cnt)(
                        lambda ji=ji, i=i: start_gather(ji + NB, R[i], G[i]))
                return 0
            lax.fori_loop(0, nrounds, rnd, 0)

            if with_deg:
                wait_deg(0)

            for r in range(NB - 1):
                def tail(r=r):
                    jt = NB * nrounds + r
                    wait_gather(jt, R[r], G[r])
                    pltpu.sync_copy(R[r], agg_sh.at[dst_v.at[jt]], add=True)
                    if with_deg:
                        def td():
                            pltpu.sync_copy(ones_v, deg_sh.at[dst_v.at[jt]],
                                            add=True)
                        pl.when((r & 1) == c)(td)
                pl.when(NB * nrounds + r < cnt)(tail)

            plsc.subcore_barrier()

        def copy_agg_out(q):
            pltpu.sync_copy(agg_sh.at[pl.ds(s * STRIPE, STRIPE)],
                            agg_hbm.at[pl.ds(s * STRIPE, STRIPE),
                                       pl.ds(q * DQ, DQ)])

        def copy_deg_out(deg_hbm):
            pltpu.sync_copy(deg_sh.at[pl.ds(s * STRIPE, STRIPE)],
                            deg_hbm.at[pl.ds(s * STRIPE, STRIPE)])

        run_pass(2 * c, True, True)

        copy_agg_out(2 * c)
        pl.when(c == 0)(lambda: copy_deg_out(deg0_hbm))
        pl.when(c == 1)(lambda: copy_deg_out(deg1_hbm))
        zero_agg_stripe()
        plsc.subcore_barrier()

        run_pass(2 * c + 1, False, False)
        copy_agg_out(2 * c + 1)

    return k(xf, edges_c)


def _tc_body(a_ref, d0_ref, d1_ref, x_ref, w1_ref, ws_ref, o_ref):
    deg = jnp.maximum(d0_ref[:, :1] + d1_ref[:, :1], 1.0)
    acc = jnp.dot(x_ref[:], ws_ref[:], preferred_element_type=jnp.float32)
    acc += jnp.dot(a_ref[:], w1_ref[:],
                   preferred_element_type=jnp.float32) / deg
    o_ref[:] = acc


def _tc_epilogue(agg, deg0, deg1, x, w1_t, w2_t):
    blk = 2000
    grid = (N_NODES // blk,)
    return pl.pallas_call(
        _tc_body,
        grid=grid,
        in_specs=[
            pl.BlockSpec((blk, D), lambda i: (i, 0)),
            pl.BlockSpec((blk, 16), lambda i: (i, 0)),
            pl.BlockSpec((blk, 16), lambda i: (i, 0)),
            pl.BlockSpec((blk, D), lambda i: (i, 0)),
            pl.BlockSpec((D, D), lambda i: (0, 0)),
            pl.BlockSpec((D, D), lambda i: (0, 0)),
        ],
        out_specs=pl.BlockSpec((blk, D), lambda i: (i, 0)),
        out_shape=jax.ShapeDtypeStruct((N_NODES, D), jnp.float32),
    )(agg, deg0, deg1, x, w1_t, w2_t)


@jax.jit
def kernel(x, edge_index, W1, W2):
    edges_c = edge_index.astype(jnp.int32).reshape(2 * NCHUNKS, CHUNK)
    xf = x.reshape(N_NODES * 4, DQ)
    agg, deg0, deg1 = _sc_aggregate(xf, edges_c)
    return _tc_epilogue(agg, deg0, deg1, x, W1.T, W2.T)
